# Initial kernel scaffold; baseline (speedup 1.0000x reference)
import jax
import jax.numpy as jnp
from jax.experimental import pallas as pl


def kernel(x, edge_index, W1, b1, Wmu, bmu, Wls, bls):
    raise NotImplementedError("write your pallas kernel here")


# speedup vs baseline: 9.3220x; 9.3220x over previous
import functools

import jax
import jax.numpy as jnp
from jax import lax
from jax.experimental import pallas as pl
from jax.experimental.pallas import tpu as pltpu
from jax.experimental.pallas import tpu_sc as plsc

N = 10000
E = 320000
NPAD = 10240
NPT = NPAD // 16
D = 128
NC = 2
NS = 16
EPAD = 327680
ROWS = EPAD // 128
CROWS = ROWS // NC
TROWS = CROWS // NS
CHUNKS = TROWS // 8
PAD_DST = N + 8
BLK = NPAD // 10
ICHUNK = NPT // 128


def _sc_mesh():
    return plsc.VectorSubcoreMesh(core_axis_name="c", subcore_axis_name="s")


def _deg_kernel(dst2d, ones2d):
    @functools.partial(
        pl.kernel,
        mesh=_sc_mesh(),
        out_type=jax.ShapeDtypeStruct((NC * NPAD, D), jnp.float32),
        scratch_types=[
            pltpu.VMEM((8, 128), jnp.int32),
            pltpu.VMEM((256, D), jnp.float32),
            pltpu.VMEM_SHARED((NPAD, D), jnp.float32),
        ],
    )
    def k(dst_h, o_h, out_h, dst_v, buf_v, deg_sh):
        c = lax.axis_index("c")
        s = lax.axis_index("s")
        rbase = s * NPT
        pltpu.sync_copy(o_h, buf_v)
        for t in range(ICHUNK):
            pltpu.sync_copy(
                buf_v.at[pl.ds(128, 128)], deg_sh.at[pl.ds(rbase + t * 128, 128)]
            )
        plsc.subcore_barrier()

        row0 = c * CROWS + s * TROWS

        def chunk(kk, carry):
            pltpu.sync_copy(dst_h.at[pl.ds(row0 + kk * 8, 8)], dst_v)
            for j in range(8):
                pltpu.sync_copy(
                    buf_v.at[pl.ds(0, 128)], deg_sh.at[dst_v.at[j]], add=True
                )
            return carry

        lax.fori_loop(0, CHUNKS, chunk, 0)
        plsc.subcore_barrier()
        for t in range(ICHUNK):
            pltpu.sync_copy(deg_sh.at[pl.ds(rbase + t * 128, 128)], buf_v.at[pl.ds(0, 128)])
            pltpu.sync_copy(
                buf_v.at[pl.ds(0, 128)],
                out_h.at[pl.ds(c * NPAD + rbase + t * 128, 128)],
            )

    return k(dst2d, ones2d)


def _prop_kernel(ytab, src2d, dst2d):
    @functools.partial(
        pl.kernel,
        mesh=_sc_mesh(),
        out_type=jax.ShapeDtypeStruct((2 * NPAD, D), jnp.float32),
        scratch_types=[
            pltpu.VMEM((8, 128), jnp.int32),
            pltpu.VMEM((8, 128), jnp.int32),
            pltpu.VMEM((256, D), jnp.float32),
            pltpu.VMEM_SHARED((NPAD, D), jnp.float32),
            pltpu.SemaphoreType.DMA,
        ],
    )
    def k(ytab_h, src_h, dst_h, out_h, src_v, dst_v, rows_v, acc_sh, sem):
        c = lax.axis_index("c")
        s = lax.axis_index("s")
        nbase = s * NPT
        for t in range(ICHUNK):
            pltpu.sync_copy(
                ytab_h.at[pl.ds(nbase + t * 128, 128)], rows_v.at[pl.ds(0, 128)]
            )
            pltpu.sync_copy(
                rows_v.at[pl.ds(0, 128)], acc_sh.at[pl.ds(nbase + t * 128, 128)]
            )
        plsc.subcore_barrier()

        row0 = c * CROWS + s * TROWS

        def chunk(kk, carry):
            r = row0 + kk * 8
            pltpu.sync_copy(src_h.at[pl.ds(r, 8)], src_v)
            pltpu.sync_copy(dst_h.at[pl.ds(r, 8)], dst_v)
            for j in range(8):
                sl = (j % 2) * 128
                pltpu.async_copy(
                    ytab_h.at[src_v.at[j]], rows_v.at[pl.ds(sl, 128)], sem
                ).wait()
                pltpu.sync_copy(
                    rows_v.at[pl.ds(sl, 128)], acc_sh.at[dst_v.at[j]], add=True
                )
            return carry

        lax.fori_loop(0, CHUNKS, chunk, 0)
        plsc.subcore_barrier()
        for t in range(ICHUNK):
            pltpu.sync_copy(
                acc_sh.at[pl.ds(nbase + t * 128, 128)], rows_v.at[pl.ds(0, 128)]
            )
            pltpu.sync_copy(
                rows_v.at[pl.ds(0, 128)],
                out_h.at[pl.ds(c * NPAD + nbase + t * 128, 128)],
            )

    return k(ytab, src2d, dst2d)


def _lin1_scale_kernel(x, W1, degp):
    def body(x_ref, w_ref, d0_ref, d1_ref, y_ref, dv_ref):
        deg = d0_ref[:, 0:1] + d1_ref[:, 0:1]
        dinv = lax.rsqrt(deg + 1.0)
        dv_ref[...] = dinv
        lin = jnp.dot(x_ref[...], w_ref[...], preferred_element_type=jnp.float32)
        y_ref[...] = lin * dinv

    return pl.pallas_call(
        body,
        grid=(10,),
        in_specs=[
            pl.BlockSpec((BLK, D), lambda i: (i, 0)),
            pl.BlockSpec((D, D), lambda i: (0, 0)),
            pl.BlockSpec((BLK, D), lambda i: (i, 0)),
            pl.BlockSpec((BLK, D), lambda i: (i, 0)),
        ],
        out_specs=[
            pl.BlockSpec((BLK, D), lambda i: (i, 0)),
            pl.BlockSpec((BLK, 1), lambda i: (i, 0)),
        ],
        out_shape=[
            jax.ShapeDtypeStruct((NPAD, D), jnp.float32),
            jax.ShapeDtypeStruct((NPAD, 1), jnp.float32),
        ],
    )(x, W1, degp[:NPAD], degp[NPAD:])


def _relu_scale_kernel(acc1, y1, dinv, b1):
    def body(a0_ref, a1_ref, y_ref, d_ref, b_ref, o_ref):
        dcol = d_ref[...]
        acc = a0_ref[...] + a1_ref[...] - y_ref[...]
        h = jnp.maximum(acc * dcol + b_ref[...], 0.0)
        o_ref[...] = h * dcol

    return pl.pallas_call(
        body,
        grid=(10,),
        in_specs=[
            pl.BlockSpec((BLK, D), lambda i: (i, 0)),
            pl.BlockSpec((BLK, D), lambda i: (i, 0)),
            pl.BlockSpec((BLK, D), lambda i: (i, 0)),
            pl.BlockSpec((BLK, 1), lambda i: (i, 0)),
            pl.BlockSpec((1, D), lambda i: (0, 0)),
        ],
        out_specs=pl.BlockSpec((BLK, D), lambda i: (i, 0)),
        out_shape=jax.ShapeDtypeStruct((NPAD, D), jnp.float32),
    )(acc1[:NPAD], acc1[NPAD:], y1, dinv, b1)


def _heads_kernel(acc2, y2, dinv, W2, b2):
    def body(a0_ref, a1_ref, y_ref, d_ref, w_ref, b_ref, mu_ref, ls_ref):
        g = (a0_ref[...] + a1_ref[...] - y_ref[...]) * d_ref[...]
        o = jnp.dot(g, w_ref[...], preferred_element_type=jnp.float32) + b_ref[...]
        mu_ref[...] = o[:, : D // 2]
        ls_ref[...] = o[:, D // 2 :]

    return pl.pallas_call(
        body,
        grid=(10,),
        in_specs=[
            pl.BlockSpec((BLK, D), lambda i: (i, 0)),
            pl.BlockSpec((BLK, D), lambda i: (i, 0)),
            pl.BlockSpec((BLK, D), lambda i: (i, 0)),
            pl.BlockSpec((BLK, 1), lambda i: (i, 0)),
            pl.BlockSpec((D, D), lambda i: (0, 0)),
            pl.BlockSpec((1, D), lambda i: (0, 0)),
        ],
        out_specs=[
            pl.BlockSpec((BLK, D // 2), lambda i: (i, 0)),
            pl.BlockSpec((BLK, D // 2), lambda i: (i, 0)),
        ],
        out_shape=[
            jax.ShapeDtypeStruct((NPAD, D // 2), jnp.float32),
            jax.ShapeDtypeStruct((NPAD, D // 2), jnp.float32),
        ],
    )(acc2[:NPAD], acc2[NPAD:], y2, dinv, W2, b2)


def kernel(x, edge_index, W1, b1, Wmu, bmu, Wls, bls):
    src = edge_index[0].astype(jnp.int32)
    dst = edge_index[1].astype(jnp.int32)
    pad = EPAD - E
    srcp = jnp.concatenate([src, jnp.zeros((pad,), jnp.int32)])
    dstp = jnp.concatenate([dst, jnp.full((pad,), PAD_DST, jnp.int32)])
    src2d = srcp.reshape(ROWS, 128)
    dst2d = dstp.reshape(ROWS, 128)
    ones2d = jnp.concatenate(
        [jnp.ones((128, D), jnp.float32), jnp.zeros((128, D), jnp.float32)]
    )
    xp = jnp.pad(x, ((0, NPAD - N), (0, 0)))
    W2 = jnp.concatenate([Wmu, Wls], axis=1)
    b2 = jnp.concatenate([bmu, bls]).reshape(1, D)
    b1r = b1.reshape(1, D)

    degp = _deg_kernel(dst2d, ones2d)
    y1, dinv = _lin1_scale_kernel(xp, W1, degp)
    acc1 = _prop_kernel(y1, src2d, dst2d)
    y2 = _relu_scale_kernel(acc1, y1, dinv, b1r)
    acc2 = _prop_kernel(y2, src2d, dst2d)
    mu, logstd = _heads_kernel(acc2, y2, dinv, W2, b2)
    return (mu[:N], logstd[:N])


# speedup vs baseline: 10.1393x; 1.0877x over previous
import functools

import jax
import jax.numpy as jnp
from jax import lax
from jax.experimental import pallas as pl
from jax.experimental.pallas import tpu as pltpu
from jax.experimental.pallas import tpu_sc as plsc

N = 10000
E = 320000
NPAD = 10240
NPT = NPAD // 16
D = 128
NC = 2
NS = 16
EPAD = 327680
ROWS = EPAD // 128
CROWS = ROWS // NC
TROWS = CROWS // NS
CHUNKS = TROWS // 8
PAD_DST = N + 8
BLK = NPAD // 10
ICHUNK = NPT // 128


def _sc_mesh():
    return plsc.VectorSubcoreMesh(core_axis_name="c", subcore_axis_name="s")


def _deg_kernel(dst2d, ones2d):
    @functools.partial(
        pl.kernel,
        mesh=_sc_mesh(),
        out_type=jax.ShapeDtypeStruct((NC * NPAD, D), jnp.float32),
        scratch_types=[
            pltpu.VMEM((8, 128), jnp.int32),
            pltpu.VMEM((256, D), jnp.float32),
            pltpu.VMEM_SHARED((NPAD, D), jnp.float32),
        ],
    )
    def k(dst_h, o_h, out_h, dst_v, buf_v, deg_sh):
        c = lax.axis_index("c")
        s = lax.axis_index("s")
        rbase = s * NPT
        pltpu.sync_copy(o_h, buf_v)
        for t in range(ICHUNK):
            pltpu.sync_copy(
                buf_v.at[pl.ds(128, 128)], deg_sh.at[pl.ds(rbase + t * 128, 128)]
            )
        plsc.subcore_barrier()

        row0 = c * CROWS + s * TROWS

        def chunk(kk, carry):
            pltpu.sync_copy(dst_h.at[pl.ds(row0 + kk * 8, 8)], dst_v)
            for j in range(8):
                pltpu.sync_copy(
                    buf_v.at[pl.ds(0, 128)], deg_sh.at[dst_v.at[j]], add=True
                )
            return carry

        lax.fori_loop(0, CHUNKS, chunk, 0)
        plsc.subcore_barrier()
        for t in range(ICHUNK):
            pltpu.sync_copy(deg_sh.at[pl.ds(rbase + t * 128, 128)], buf_v.at[pl.ds(0, 128)])
            pltpu.sync_copy(
                buf_v.at[pl.ds(0, 128)],
                out_h.at[pl.ds(c * NPAD + rbase + t * 128, 128)],
            )

    return k(dst2d, ones2d)


def _prop_kernel(ytab, src2d, dst2d):
    @functools.partial(
        pl.kernel,
        mesh=_sc_mesh(),
        out_type=jax.ShapeDtypeStruct((2 * NPAD, D), jnp.float32),
        scratch_types=[
            pltpu.VMEM((8, 128), jnp.int32),
            pltpu.VMEM((8, 128), jnp.int32),
            pltpu.VMEM((256, D), jnp.float32),
            pltpu.VMEM_SHARED((NPAD, D), jnp.float32),
            pltpu.SemaphoreType.DMA,
        ],
    )
    def k(ytab_h, src_h, dst_h, out_h, src_v, dst_v, rows_v, acc_sh, sem):
        c = lax.axis_index("c")
        s = lax.axis_index("s")
        nbase = s * NPT
        for t in range(ICHUNK):
            pltpu.sync_copy(
                ytab_h.at[pl.ds(nbase + t * 128, 128)], rows_v.at[pl.ds(0, 128)]
            )
            pltpu.sync_copy(
                rows_v.at[pl.ds(0, 128)], acc_sh.at[pl.ds(nbase + t * 128, 128)]
            )
        plsc.subcore_barrier()

        row0 = c * CROWS + s * TROWS

        def chunk(kk, carry):
            r = row0 + kk * 8
            pltpu.sync_copy(src_h.at[pl.ds(r, 8)], src_v)
            pltpu.sync_copy(dst_h.at[pl.ds(r, 8)], dst_v)
            hs = [None] * 8
            hs[0] = pltpu.async_copy(
                ytab_h.at[src_v.at[0]], rows_v.at[pl.ds(0, 128)], sem
            )
            for j in range(8):
                if j + 1 < 8:
                    nsl = ((j + 1) % 2) * 128
                    hs[j + 1] = pltpu.async_copy(
                        ytab_h.at[src_v.at[j + 1]], rows_v.at[pl.ds(nsl, 128)], sem
                    )
                hs[j].wait()
                pltpu.sync_copy(
                    rows_v.at[pl.ds((j % 2) * 128, 128)],
                    acc_sh.at[dst_v.at[j]],
                    add=True,
                )
            return carry

        lax.fori_loop(0, CHUNKS, chunk, 0)
        plsc.subcore_barrier()
        for t in range(ICHUNK):
            pltpu.sync_copy(
                acc_sh.at[pl.ds(nbase + t * 128, 128)], rows_v.at[pl.ds(0, 128)]
            )
            pltpu.sync_copy(
                rows_v.at[pl.ds(0, 128)],
                out_h.at[pl.ds(c * NPAD + nbase + t * 128, 128)],
            )

    return k(ytab, src2d, dst2d)


def _lin1_scale_kernel(x, W1, degp):
    def body(x_ref, w_ref, d0_ref, d1_ref, y_ref, dv_ref):
        deg = d0_ref[:, 0:1] + d1_ref[:, 0:1]
        dinv = lax.rsqrt(deg + 1.0)
        dv_ref[...] = dinv
        lin = jnp.dot(x_ref[...], w_ref[...], preferred_element_type=jnp.float32)
        y_ref[...] = lin * dinv

    return pl.pallas_call(
        body,
        grid=(10,),
        in_specs=[
            pl.BlockSpec((BLK, D), lambda i: (i, 0)),
            pl.BlockSpec((D, D), lambda i: (0, 0)),
            pl.BlockSpec((BLK, D), lambda i: (i, 0)),
            pl.BlockSpec((BLK, D), lambda i: (i, 0)),
        ],
        out_specs=[
            pl.BlockSpec((BLK, D), lambda i: (i, 0)),
            pl.BlockSpec((BLK, 1), lambda i: (i, 0)),
        ],
        out_shape=[
            jax.ShapeDtypeStruct((NPAD, D), jnp.float32),
            jax.ShapeDtypeStruct((NPAD, 1), jnp.float32),
        ],
    )(x, W1, degp[:NPAD], degp[NPAD:])


def _relu_scale_kernel(acc1, y1, dinv, b1):
    def body(a0_ref, a1_ref, y_ref, d_ref, b_ref, o_ref):
        dcol = d_ref[...]
        acc = a0_ref[...] + a1_ref[...] - y_ref[...]
        h = jnp.maximum(acc * dcol + b_ref[...], 0.0)
        o_ref[...] = h * dcol

    return pl.pallas_call(
        body,
        grid=(10,),
        in_specs=[
            pl.BlockSpec((BLK, D), lambda i: (i, 0)),
            pl.BlockSpec((BLK, D), lambda i: (i, 0)),
            pl.BlockSpec((BLK, D), lambda i: (i, 0)),
            pl.BlockSpec((BLK, 1), lambda i: (i, 0)),
            pl.BlockSpec((1, D), lambda i: (0, 0)),
        ],
        out_specs=pl.BlockSpec((BLK, D), lambda i: (i, 0)),
        out_shape=jax.ShapeDtypeStruct((NPAD, D), jnp.float32),
    )(acc1[:NPAD], acc1[NPAD:], y1, dinv, b1)


def _heads_kernel(acc2, y2, dinv, W2, b2):
    def body(a0_ref, a1_ref, y_ref, d_ref, w_ref, b_ref, mu_ref, ls_ref):
        g = (a0_ref[...] + a1_ref[...] - y_ref[...]) * d_ref[...]
        o = jnp.dot(g, w_ref[...], preferred_element_type=jnp.float32) + b_ref[...]
        mu_ref[...] = o[:, : D // 2]
        ls_ref[...] = o[:, D // 2 :]

    return pl.pallas_call(
        body,
        grid=(10,),
        in_specs=[
            pl.BlockSpec((BLK, D), lambda i: (i, 0)),
            pl.BlockSpec((BLK, D), lambda i: (i, 0)),
            pl.BlockSpec((BLK, D), lambda i: (i, 0)),
            pl.BlockSpec((BLK, 1), lambda i: (i, 0)),
            pl.BlockSpec((D, D), lambda i: (0, 0)),
            pl.BlockSpec((1, D), lambda i: (0, 0)),
        ],
        out_specs=[
            pl.BlockSpec((BLK, D // 2), lambda i: (i, 0)),
            pl.BlockSpec((BLK, D // 2), lambda i: (i, 0)),
        ],
        out_shape=[
            jax.ShapeDtypeStruct((NPAD, D // 2), jnp.float32),
            jax.ShapeDtypeStruct((NPAD, D // 2), jnp.float32),
        ],
    )(acc2[:NPAD], acc2[NPAD:], y2, dinv, W2, b2)


def kernel(x, edge_index, W1, b1, Wmu, bmu, Wls, bls):
    src = edge_index[0].astype(jnp.int32)
    dst = edge_index[1].astype(jnp.int32)
    pad = EPAD - E
    srcp = jnp.concatenate([src, jnp.zeros((pad,), jnp.int32)])
    pad_sink = PAD_DST + (jnp.arange(pad, dtype=jnp.int32) % 128)
    dstp = jnp.concatenate([dst, pad_sink])
    src2d = srcp.reshape(ROWS, 128)
    dst2d = dstp.reshape(ROWS, 128)
    ones2d = jnp.concatenate(
        [jnp.ones((128, D), jnp.float32), jnp.zeros((128, D), jnp.float32)]
    )
    xp = jnp.pad(x, ((0, NPAD - N), (0, 0)))
    W2 = jnp.concatenate([Wmu, Wls], axis=1)
    b2 = jnp.concatenate([bmu, bls]).reshape(1, D)
    b1r = b1.reshape(1, D)

    degp = _deg_kernel(dst2d, ones2d)
    y1, dinv = _lin1_scale_kernel(xp, W1, degp)
    acc1 = _prop_kernel(y1, src2d, dst2d)
    y2 = _relu_scale_kernel(acc1, y1, dinv, b1r)
    acc2 = _prop_kernel(y2, src2d, dst2d)
    mu, logstd = _heads_kernel(acc2, y2, dinv, W2, b2)
    return (mu[:N], logstd[:N])


# speedup vs baseline: 27.3699x; 2.6994x over previous
import functools

import jax
import jax.numpy as jnp
from jax import lax
from jax.experimental import pallas as pl
from jax.experimental.pallas import tpu as pltpu
from jax.experimental.pallas import tpu_sc as plsc

N = 10000
E = 320000
NPAD = 10240
NPT = NPAD // 16
D = 128
NC = 2
NS = 16
EPAD = 327680
ROWS = EPAD // 128
CROWS = ROWS // NC
TROWS = CROWS // NS
CHUNKS = TROWS // 8
PAD_DST = N + 8
BLK = NPAD // 10
ICHUNK = NPT // 128


def _sc_mesh():
    return plsc.VectorSubcoreMesh(core_axis_name="c", subcore_axis_name="s")


def _deg_kernel(dst2d, ones2d):
    @functools.partial(
        pl.kernel,
        mesh=_sc_mesh(),
        out_type=jax.ShapeDtypeStruct((NC * NPAD, D), jnp.float32),
        scratch_types=[
            pltpu.VMEM((8, 128), jnp.int32),
            pltpu.VMEM((256, D), jnp.float32),
            pltpu.VMEM_SHARED((NPAD, D), jnp.float32),
        ],
    )
    def k(dst_h, o_h, out_h, dst_v, buf_v, deg_sh):
        c = lax.axis_index("c")
        s = lax.axis_index("s")
        rbase = s * NPT
        pltpu.sync_copy(o_h, buf_v)
        for t in range(ICHUNK):
            pltpu.sync_copy(
                buf_v.at[pl.ds(128, 128)], deg_sh.at[pl.ds(rbase + t * 128, 128)]
            )
        plsc.subcore_barrier()

        row0 = c * CROWS + s * TROWS

        def chunk(kk, carry):
            pltpu.sync_copy(dst_h.at[pl.ds(row0 + kk * 8, 8)], dst_v)
            for j in range(8):
                pltpu.sync_copy(
                    buf_v.at[pl.ds(0, 128)], deg_sh.at[dst_v.at[j]], add=True
                )
            return carry

        lax.fori_loop(0, CHUNKS, chunk, 0)
        plsc.subcore_barrier()
        for t in range(ICHUNK):
            pltpu.sync_copy(deg_sh.at[pl.ds(rbase + t * 128, 128)], buf_v.at[pl.ds(0, 128)])
            pltpu.sync_copy(
                buf_v.at[pl.ds(0, 128)],
                out_h.at[pl.ds(c * NPAD + rbase + t * 128, 128)],
            )

    return k(dst2d, ones2d)


def _prop_kernel(ytab, src2d, dst2d):
    @functools.partial(
        pl.kernel,
        mesh=_sc_mesh(),
        out_type=jax.ShapeDtypeStruct((2 * NPAD, D), jnp.float32),
        scratch_types=[
            pltpu.VMEM((8, 128), jnp.int32),
            pltpu.VMEM((8, 128), jnp.int32),
            pltpu.VMEM((256, D), jnp.float32),
            pltpu.VMEM_SHARED((NPAD, D), jnp.float32),
            pltpu.SemaphoreType.DMA,
        ],
    )
    def k(ytab_h, src_h, dst_h, out_h, src_v, dst_v, rows_v, acc_sh, sem):
        c = lax.axis_index("c")
        s = lax.axis_index("s")
        nbase = s * NPT
        for t in range(ICHUNK):
            pltpu.sync_copy(
                ytab_h.at[pl.ds(nbase + t * 128, 128)], rows_v.at[pl.ds(0, 128)]
            )
            pltpu.sync_copy(
                rows_v.at[pl.ds(0, 128)], acc_sh.at[pl.ds(nbase + t * 128, 128)]
            )
        plsc.subcore_barrier()

        row0 = c * CROWS + s * TROWS

        def chunk(kk, carry):
            r = row0 + kk * 8
            pltpu.sync_copy(src_h.at[pl.ds(r, 8)], src_v)
            pltpu.sync_copy(dst_h.at[pl.ds(r, 8)], dst_v)
            hs = [None] * 8
            hs[0] = pltpu.async_copy(
                ytab_h.at[src_v.at[0]], rows_v.at[pl.ds(0, 128)], sem
            )
            for j in range(8):
                if j + 1 < 8:
                    nsl = ((j + 1) % 2) * 128
                    hs[j + 1] = pltpu.async_copy(
                        ytab_h.at[src_v.at[j + 1]], rows_v.at[pl.ds(nsl, 128)], sem
                    )
                hs[j].wait()
                pltpu.sync_copy(
                    rows_v.at[pl.ds((j % 2) * 128, 128)],
                    acc_sh.at[dst_v.at[j]],
                    add=True,
                )
            return carry

        lax.fori_loop(0, CHUNKS, chunk, 0)
        plsc.subcore_barrier()
        for t in range(ICHUNK):
            pltpu.sync_copy(
                acc_sh.at[pl.ds(nbase + t * 128, 128)], rows_v.at[pl.ds(0, 128)]
            )
            pltpu.sync_copy(
                rows_v.at[pl.ds(0, 128)],
                out_h.at[pl.ds(c * NPAD + nbase + t * 128, 128)],
            )

    return k(ytab, src2d, dst2d)


def _lin1_scale_kernel(x, W1, degp):
    def body(x_ref, w_ref, d0_ref, d1_ref, y_ref, dv_ref):
        deg = d0_ref[:, 0:1] + d1_ref[:, 0:1]
        dinv = lax.rsqrt(deg + 1.0)
        dv_ref[...] = dinv
        lin = jnp.dot(x_ref[...], w_ref[...], preferred_element_type=jnp.float32)
        y_ref[...] = lin * dinv

    return pl.pallas_call(
        body,
        grid=(10,),
        in_specs=[
            pl.BlockSpec((BLK, D), lambda i: (i, 0)),
            pl.BlockSpec((D, D), lambda i: (0, 0)),
            pl.BlockSpec((BLK, D), lambda i: (i, 0)),
            pl.BlockSpec((BLK, D), lambda i: (i, 0)),
        ],
        out_specs=[
            pl.BlockSpec((BLK, D), lambda i: (i, 0)),
            pl.BlockSpec((BLK, 1), lambda i: (i, 0)),
        ],
        out_shape=[
            jax.ShapeDtypeStruct((NPAD, D), jnp.float32),
            jax.ShapeDtypeStruct((NPAD, 1), jnp.float32),
        ],
    )(x, W1, degp[:NPAD], degp[NPAD:])


def _relu_scale_kernel(acc1, y1, dinv, b1):
    def body(a0_ref, a1_ref, y_ref, d_ref, b_ref, o_ref):
        dcol = d_ref[...]
        acc = a0_ref[...] + a1_ref[...] - y_ref[...]
        h = jnp.maximum(acc * dcol + b_ref[...], 0.0)
        o_ref[...] = h * dcol

    return pl.pallas_call(
        body,
        grid=(10,),
        in_specs=[
            pl.BlockSpec((BLK, D), lambda i: (i, 0)),
            pl.BlockSpec((BLK, D), lambda i: (i, 0)),
            pl.BlockSpec((BLK, D), lambda i: (i, 0)),
            pl.BlockSpec((BLK, 1), lambda i: (i, 0)),
            pl.BlockSpec((1, D), lambda i: (0, 0)),
        ],
        out_specs=pl.BlockSpec((BLK, D), lambda i: (i, 0)),
        out_shape=jax.ShapeDtypeStruct((NPAD, D), jnp.float32),
    )(acc1[:NPAD], acc1[NPAD:], y1, dinv, b1)


def _heads_kernel(acc2, y2, dinv, W2, b2):
    def body(a0_ref, a1_ref, y_ref, d_ref, w_ref, b_ref, mu_ref, ls_ref):
        g = (a0_ref[...] + a1_ref[...] - y_ref[...]) * d_ref[...]
        o = jnp.dot(g, w_ref[...], preferred_element_type=jnp.float32) + b_ref[...]
        mu_ref[...] = o[:, : D // 2]
        ls_ref[...] = o[:, D // 2 :]

    return pl.pallas_call(
        body,
        grid=(10,),
        in_specs=[
            pl.BlockSpec((BLK, D), lambda i: (i, 0)),
            pl.BlockSpec((BLK, D), lambda i: (i, 0)),
            pl.BlockSpec((BLK, D), lambda i: (i, 0)),
            pl.BlockSpec((BLK, 1), lambda i: (i, 0)),
            pl.BlockSpec((D, D), lambda i: (0, 0)),
            pl.BlockSpec((1, D), lambda i: (0, 0)),
        ],
        out_specs=[
            pl.BlockSpec((BLK, D // 2), lambda i: (i, 0)),
            pl.BlockSpec((BLK, D // 2), lambda i: (i, 0)),
        ],
        out_shape=[
            jax.ShapeDtypeStruct((NPAD, D // 2), jnp.float32),
            jax.ShapeDtypeStruct((NPAD, D // 2), jnp.float32),
        ],
    )(acc2[:NPAD], acc2[NPAD:], y2, dinv, W2, b2)


def kernel(x, edge_index, W1, b1, Wmu, bmu, Wls, bls):
    src = edge_index[0].astype(jnp.int32)
    dst = edge_index[1].astype(jnp.int32)
    pad = EPAD - E
    pad_lane = jnp.arange(pad, dtype=jnp.int32) % 128
    srcp = jnp.concatenate([src, pad_lane * 64])
    dstp = jnp.concatenate([dst, PAD_DST + pad_lane])
    src2d = srcp.reshape(ROWS, 128)
    dst2d = dstp.reshape(ROWS, 128)
    ones2d = jnp.concatenate(
        [jnp.ones((128, D), jnp.float32), jnp.zeros((128, D), jnp.float32)]
    )
    xp = jnp.pad(x, ((0, NPAD - N), (0, 0)))
    W2 = jnp.concatenate([Wmu, Wls], axis=1)
    b2 = jnp.concatenate([bmu, bls]).reshape(1, D)
    b1r = b1.reshape(1, D)

    degp = _deg_kernel(dst2d, ones2d)
    y1, dinv = _lin1_scale_kernel(xp, W1, degp)
    acc1 = _prop_kernel(y1, src2d, dst2d)
    y2 = _relu_scale_kernel(acc1, y1, dinv, b1r)
    acc2 = _prop_kernel(y2, src2d, dst2d)
    mu, logstd = _heads_kernel(acc2, y2, dinv, W2, b2)
    return (mu[:N], logstd[:N])


# speedup vs baseline: 32.1637x; 1.1751x over previous
import functools

import jax
import jax.numpy as jnp
from jax import lax
from jax.experimental import pallas as pl
from jax.experimental.pallas import tpu as pltpu
from jax.experimental.pallas import tpu_sc as plsc

N = 10000
E = 320000
NPAD = 10240
NPT = NPAD // 16
D = 128
NC = 2
NS = 16
EPAD = 327680
ROWS = EPAD // 128
CROWS = ROWS // NC
TROWS = CROWS // NS
CHUNKS = TROWS // 8
PAD_DST = N + 8
BLK = NPAD // 10
ICHUNK = NPT // 128


def _sc_mesh():
    return plsc.VectorSubcoreMesh(core_axis_name="c", subcore_axis_name="s")


def _deg_kernel(dst2d, zeros2d, iota2d):
    @functools.partial(
        pl.kernel,
        mesh=_sc_mesh(),
        out_type=jax.ShapeDtypeStruct((NC * 128, D), jnp.float32),
        scratch_types=[
            pltpu.VMEM((TROWS, 128), jnp.int32),
            pltpu.VMEM((1, 128), jnp.int32),
            pltpu.VMEM((128, D), jnp.float32),
            pltpu.VMEM((16, D), jnp.float32),
            pltpu.VMEM_SHARED((128, D), jnp.float32),
        ],
        compiler_params=pltpu.CompilerParams(needs_layout_passes=False),
    )
    def k(dst_h, z_h, iota_h, out_h, dst_v, idx_v, hist_v, wb_v, deg_sh):
        c = lax.axis_index("c")
        s = lax.axis_index("s")
        pltpu.sync_copy(z_h, hist_v)
        pltpu.sync_copy(iota_h, idx_v)

        @pl.when(s == 0)
        def _():
            pltpu.sync_copy(hist_v, deg_sh)

        plsc.subcore_barrier()

        row0 = c * CROWS + s * TROWS
        pltpu.sync_copy(dst_h.at[pl.ds(row0, TROWS)], dst_v)
        ones16 = jnp.full((16,), 1.0, dtype=jnp.float32)

        def chunk(kk, carry):
            for j in range(8):
                for t in range(8):
                    v = dst_v[kk * 8 + j, pl.ds(t * 16, 16)]
                    hi = jnp.right_shift(v, 7)
                    lo = jnp.bitwise_and(v, 127)
                    plsc.addupdate_scatter(hist_v, [hi, lo], ones16)
            return carry

        lax.fori_loop(0, CHUNKS, chunk, 0)
        pltpu.sync_copy(hist_v, deg_sh.at[idx_v.at[0]], add=True)
        plsc.subcore_barrier()

        @pl.when(s < 8)
        def _():
            pltpu.sync_copy(deg_sh.at[pl.ds(s * 16, 16)], wb_v)
            pltpu.sync_copy(wb_v, out_h.at[pl.ds(c * 128 + s * 16, 16)])

    return k(dst2d, zeros2d, iota2d)


def _prop_kernel(ytab, src2d, dst2d):
    @functools.partial(
        pl.kernel,
        mesh=_sc_mesh(),
        out_type=jax.ShapeDtypeStruct((2 * NPAD, D), jnp.float32),
        scratch_types=[
            pltpu.VMEM((8, 128), jnp.int32),
            pltpu.VMEM((8, 128), jnp.int32),
            pltpu.VMEM((256, D), jnp.float32),
            pltpu.VMEM_SHARED((NPAD, D), jnp.float32),
            pltpu.SemaphoreType.DMA,
        ],
    )
    def k(ytab_h, src_h, dst_h, out_h, src_v, dst_v, rows_v, acc_sh, sem):
        c = lax.axis_index("c")
        s = lax.axis_index("s")
        nbase = s * NPT
        for t in range(ICHUNK):
            pltpu.sync_copy(
                ytab_h.at[pl.ds(nbase + t * 128, 128)], rows_v.at[pl.ds(0, 128)]
            )
            pltpu.sync_copy(
                rows_v.at[pl.ds(0, 128)], acc_sh.at[pl.ds(nbase + t * 128, 128)]
            )
        plsc.subcore_barrier()

        row0 = c * CROWS + s * TROWS

        def chunk(kk, carry):
            r = row0 + kk * 8
            pltpu.sync_copy(src_h.at[pl.ds(r, 8)], src_v)
            pltpu.sync_copy(dst_h.at[pl.ds(r, 8)], dst_v)
            hs = [None] * 8
            hs[0] = pltpu.async_copy(
                ytab_h.at[src_v.at[0]], rows_v.at[pl.ds(0, 128)], sem
            )
            for j in range(8):
                if j + 1 < 8:
                    nsl = ((j + 1) % 2) * 128
                    hs[j + 1] = pltpu.async_copy(
                        ytab_h.at[src_v.at[j + 1]], rows_v.at[pl.ds(nsl, 128)], sem
                    )
                hs[j].wait()
                pltpu.sync_copy(
                    rows_v.at[pl.ds((j % 2) * 128, 128)],
                    acc_sh.at[dst_v.at[j]],
                    add=True,
                )
            return carry

        lax.fori_loop(0, CHUNKS, chunk, 0)
        plsc.subcore_barrier()
        for t in range(ICHUNK):
            pltpu.sync_copy(
                acc_sh.at[pl.ds(nbase + t * 128, 128)], rows_v.at[pl.ds(0, 128)]
            )
            pltpu.sync_copy(
                rows_v.at[pl.ds(0, 128)],
                out_h.at[pl.ds(c * NPAD + nbase + t * 128, 128)],
            )

    return k(ytab, src2d, dst2d)


def _lin1_scale_kernel(x, W1, deg0, deg1):
    def body(x_ref, w_ref, d0_ref, d1_ref, y_ref, dv_ref):
        deg = d0_ref[...] + d1_ref[...]
        dinv = lax.rsqrt(deg + 1.0)
        dv_ref[...] = dinv
        lin = jnp.dot(x_ref[...], w_ref[...], preferred_element_type=jnp.float32)
        y_ref[...] = lin * dinv

    return pl.pallas_call(
        body,
        grid=(10,),
        in_specs=[
            pl.BlockSpec((BLK, D), lambda i: (i, 0)),
            pl.BlockSpec((D, D), lambda i: (0, 0)),
            pl.BlockSpec((BLK, 1), lambda i: (i, 0)),
            pl.BlockSpec((BLK, 1), lambda i: (i, 0)),
        ],
        out_specs=[
            pl.BlockSpec((BLK, D), lambda i: (i, 0)),
            pl.BlockSpec((BLK, 1), lambda i: (i, 0)),
        ],
        out_shape=[
            jax.ShapeDtypeStruct((NPAD, D), jnp.float32),
            jax.ShapeDtypeStruct((NPAD, 1), jnp.float32),
        ],
    )(x, W1, deg0, deg1)


def _relu_scale_kernel(acc1, y1, dinv, b1):
    def body(a0_ref, a1_ref, y_ref, d_ref, b_ref, o_ref):
        dcol = d_ref[...]
        acc = a0_ref[...] + a1_ref[...] - y_ref[...]
        h = jnp.maximum(acc * dcol + b_ref[...], 0.0)
        o_ref[...] = h * dcol

    return pl.pallas_call(
        body,
        grid=(10,),
        in_specs=[
            pl.BlockSpec((BLK, D), lambda i: (i, 0)),
            pl.BlockSpec((BLK, D), lambda i: (i, 0)),
            pl.BlockSpec((BLK, D), lambda i: (i, 0)),
            pl.BlockSpec((BLK, 1), lambda i: (i, 0)),
            pl.BlockSpec((1, D), lambda i: (0, 0)),
        ],
        out_specs=pl.BlockSpec((BLK, D), lambda i: (i, 0)),
        out_shape=jax.ShapeDtypeStruct((NPAD, D), jnp.float32),
    )(acc1[:NPAD], acc1[NPAD:], y1, dinv, b1)


def _heads_kernel(acc2, y2, dinv, W2, b2):
    def body(a0_ref, a1_ref, y_ref, d_ref, w_ref, b_ref, mu_ref, ls_ref):
        g = (a0_ref[...] + a1_ref[...] - y_ref[...]) * d_ref[...]
        o = jnp.dot(g, w_ref[...], preferred_element_type=jnp.float32) + b_ref[...]
        mu_ref[...] = o[:, : D // 2]
        ls_ref[...] = o[:, D // 2 :]

    return pl.pallas_call(
        body,
        grid=(10,),
        in_specs=[
            pl.BlockSpec((BLK, D), lambda i: (i, 0)),
            pl.BlockSpec((BLK, D), lambda i: (i, 0)),
            pl.BlockSpec((BLK, D), lambda i: (i, 0)),
            pl.BlockSpec((BLK, 1), lambda i: (i, 0)),
            pl.BlockSpec((D, D), lambda i: (0, 0)),
            pl.BlockSpec((1, D), lambda i: (0, 0)),
        ],
        out_specs=[
            pl.BlockSpec((BLK, D // 2), lambda i: (i, 0)),
            pl.BlockSpec((BLK, D // 2), lambda i: (i, 0)),
        ],
        out_shape=[
            jax.ShapeDtypeStruct((NPAD, D // 2), jnp.float32),
            jax.ShapeDtypeStruct((NPAD, D // 2), jnp.float32),
        ],
    )(acc2[:NPAD], acc2[NPAD:], y2, dinv, W2, b2)


def kernel(x, edge_index, W1, b1, Wmu, bmu, Wls, bls):
    src = edge_index[0].astype(jnp.int32)
    dst = edge_index[1].astype(jnp.int32)
    pad = EPAD - E
    pad_lane = jnp.arange(pad, dtype=jnp.int32) % 128
    srcp = jnp.concatenate([src, pad_lane * 64])
    dstp = jnp.concatenate([dst, PAD_DST + pad_lane])
    src2d = srcp.reshape(ROWS, 128)
    dst2d = dstp.reshape(ROWS, 128)
    zeros2d = jnp.zeros((128, D), jnp.float32)
    iota2d = jnp.arange(128, dtype=jnp.int32).reshape(1, 128)
    xp = jnp.pad(x, ((0, NPAD - N), (0, 0)))
    W2 = jnp.concatenate([Wmu, Wls], axis=1)
    b2 = jnp.concatenate([bmu, bls]).reshape(1, D)
    b1r = b1.reshape(1, D)

    degp = _deg_kernel(dst2d, zeros2d, iota2d)
    deg0 = degp[:128].reshape(-1)[:NPAD].reshape(NPAD, 1)
    deg1 = degp[128:].reshape(-1)[:NPAD].reshape(NPAD, 1)
    y1, dinv = _lin1_scale_kernel(xp, W1, deg0, deg1)
    acc1 = _prop_kernel(y1, src2d, dst2d)
    y2 = _relu_scale_kernel(acc1, y1, dinv, b1r)
    acc2 = _prop_kernel(y2, src2d, dst2d)
    mu, logstd = _heads_kernel(acc2, y2, dinv, W2, b2)
    return (mu[:N], logstd[:N])


# speedup vs baseline: 33.5672x; 1.0436x over previous
import functools

import jax
import jax.numpy as jnp
from jax import lax
from jax.experimental import pallas as pl
from jax.experimental.pallas import tpu as pltpu
from jax.experimental.pallas import tpu_sc as plsc

N = 10000
E = 320000
NPAD = 10240
NPT = NPAD // 16
D = 128
NC = 2
NS = 16
EPAD = 327680
ROWS = EPAD // 128
CROWS = ROWS // NC
TROWS = CROWS // NS
CHUNKS = TROWS // 8
PAD_DST = N + 8
BLK = NPAD // 10
ICHUNK = NPT // 128


def _sc_mesh():
    return plsc.VectorSubcoreMesh(core_axis_name="c", subcore_axis_name="s")


def _deg_kernel(dst2d, zeros2d, iota2d):
    @functools.partial(
        pl.kernel,
        mesh=_sc_mesh(),
        out_type=jax.ShapeDtypeStruct((NC * 128, D), jnp.float32),
        scratch_types=[
            pltpu.VMEM((TROWS, 128), jnp.int32),
            pltpu.VMEM((1, 128), jnp.int32),
            pltpu.VMEM((128, D), jnp.float32),
            pltpu.VMEM((16, D), jnp.float32),
            pltpu.VMEM_SHARED((128, D), jnp.float32),
        ],
        compiler_params=pltpu.CompilerParams(needs_layout_passes=False),
    )
    def k(dst_h, z_h, iota_h, out_h, dst_v, idx_v, hist_v, wb_v, deg_sh):
        c = lax.axis_index("c")
        s = lax.axis_index("s")
        pltpu.sync_copy(z_h, hist_v)
        pltpu.sync_copy(iota_h, idx_v)

        @pl.when(s == 0)
        def _():
            pltpu.sync_copy(hist_v, deg_sh)

        plsc.subcore_barrier()

        row0 = c * CROWS + s * TROWS
        pltpu.sync_copy(dst_h.at[pl.ds(row0, TROWS)], dst_v)
        ones16 = jnp.full((16,), 1.0, dtype=jnp.float32)

        def chunk(kk, carry):
            for j in range(8):
                for t in range(8):
                    v = dst_v[kk * 8 + j, pl.ds(t * 16, 16)]
                    hi = jnp.right_shift(v, 7)
                    lo = jnp.bitwise_and(v, 127)
                    plsc.addupdate_scatter(hist_v, [hi, lo], ones16)
            return carry

        lax.fori_loop(0, CHUNKS, chunk, 0)
        pltpu.sync_copy(hist_v, deg_sh.at[idx_v.at[0]], add=True)
        plsc.subcore_barrier()

        @pl.when(s < 8)
        def _():
            pltpu.sync_copy(deg_sh.at[pl.ds(s * 16, 16)], wb_v)
            pltpu.sync_copy(wb_v, out_h.at[pl.ds(c * 128 + s * 16, 16)])

    return k(dst2d, zeros2d, iota2d)


def _prop_kernel(ytab, src2d, dst2d):
    @functools.partial(
        pl.kernel,
        mesh=_sc_mesh(),
        out_type=jax.ShapeDtypeStruct((2 * NPAD, D), jnp.float32),
        scratch_types=[
            pltpu.VMEM((16, 128), jnp.int32),
            pltpu.VMEM((16, 128), jnp.int32),
            pltpu.VMEM((256, D), jnp.float32),
            pltpu.VMEM_SHARED((NPAD, D), jnp.float32),
            pltpu.SemaphoreType.DMA,
            pltpu.SemaphoreType.DMA,
        ],
    )
    def k(ytab_h, src_h, dst_h, out_h, src_v, dst_v, rows_v, acc_sh, sem, isem):
        c = lax.axis_index("c")
        s = lax.axis_index("s")
        nbase = s * NPT
        for t in range(ICHUNK):
            pltpu.sync_copy(
                ytab_h.at[pl.ds(nbase + t * 128, 128)], rows_v.at[pl.ds(0, 128)]
            )
            pltpu.sync_copy(
                rows_v.at[pl.ds(0, 128)], acc_sh.at[pl.ds(nbase + t * 128, 128)]
            )
        plsc.subcore_barrier()

        row0 = c * CROWS + s * TROWS
        ihs = [None] * (CHUNKS * 2)
        ihs[0] = pltpu.async_copy(src_h.at[pl.ds(row0, 8)], src_v.at[pl.ds(0, 8)], isem)
        ihs[1] = pltpu.async_copy(dst_h.at[pl.ds(row0, 8)], dst_v.at[pl.ds(0, 8)], isem)
        hs = [None] * TROWS
        for kk in range(CHUNKS):
            ib = (kk % 2) * 8
            if kk + 1 < CHUNKS:
                nb = ((kk + 1) % 2) * 8
                r = row0 + (kk + 1) * 8
                ihs[2 * kk + 2] = pltpu.async_copy(
                    src_h.at[pl.ds(r, 8)], src_v.at[pl.ds(nb, 8)], isem
                )
                ihs[2 * kk + 3] = pltpu.async_copy(
                    dst_h.at[pl.ds(r, 8)], dst_v.at[pl.ds(nb, 8)], isem
                )
            ihs[2 * kk].wait()
            ihs[2 * kk + 1].wait()
            for j in range(8):
                g = kk * 8 + j
                if j == 0:
                    hs[g] = pltpu.async_copy(
                        ytab_h.at[src_v.at[ib]], rows_v.at[pl.ds((g % 2) * 128, 128)], sem
                    )
                if g + 1 < TROWS and j < 7:
                    nsl = ((g + 1) % 2) * 128
                    hs[g + 1] = pltpu.async_copy(
                        ytab_h.at[src_v.at[ib + j + 1]], rows_v.at[pl.ds(nsl, 128)], sem
                    )
                hs[g].wait()
                pltpu.sync_copy(
                    rows_v.at[pl.ds((g % 2) * 128, 128)],
                    acc_sh.at[dst_v.at[ib + j]],
                    add=True,
                )
        plsc.subcore_barrier()
        for t in range(ICHUNK):
            pltpu.sync_copy(
                acc_sh.at[pl.ds(nbase + t * 128, 128)], rows_v.at[pl.ds(0, 128)]
            )
            pltpu.sync_copy(
                rows_v.at[pl.ds(0, 128)],
                out_h.at[pl.ds(c * NPAD + nbase + t * 128, 128)],
            )

    return k(ytab, src2d, dst2d)


def _lin1_scale_kernel(x, W1, deg0, deg1):
    def body(x_ref, w_ref, d0_ref, d1_ref, y_ref, dv_ref):
        deg = d0_ref[...] + d1_ref[...]
        dinv = lax.rsqrt(deg + 1.0)
        dv_ref[...] = dinv
        lin = jnp.dot(x_ref[...], w_ref[...], preferred_element_type=jnp.float32)
        y_ref[...] = lin * dinv

    return pl.pallas_call(
        body,
        grid=(10,),
        in_specs=[
            pl.BlockSpec((BLK, D), lambda i: (i, 0)),
            pl.BlockSpec((D, D), lambda i: (0, 0)),
            pl.BlockSpec((BLK, 1), lambda i: (i, 0)),
            pl.BlockSpec((BLK, 1), lambda i: (i, 0)),
        ],
        out_specs=[
            pl.BlockSpec((BLK, D), lambda i: (i, 0)),
            pl.BlockSpec((BLK, 1), lambda i: (i, 0)),
        ],
        out_shape=[
            jax.ShapeDtypeStruct((NPAD, D), jnp.float32),
            jax.ShapeDtypeStruct((NPAD, 1), jnp.float32),
        ],
    )(x, W1, deg0, deg1)


def _relu_scale_kernel(acc1, y1, dinv, b1):
    def body(a0_ref, a1_ref, y_ref, d_ref, b_ref, o_ref):
        dcol = d_ref[...]
        acc = a0_ref[...] + a1_ref[...] - y_ref[...]
        h = jnp.maximum(acc * dcol + b_ref[...], 0.0)
        o_ref[...] = h * dcol

    return pl.pallas_call(
        body,
        grid=(10,),
        in_specs=[
            pl.BlockSpec((BLK, D), lambda i: (i, 0)),
            pl.BlockSpec((BLK, D), lambda i: (i, 0)),
            pl.BlockSpec((BLK, D), lambda i: (i, 0)),
            pl.BlockSpec((BLK, 1), lambda i: (i, 0)),
            pl.BlockSpec((1, D), lambda i: (0, 0)),
        ],
        out_specs=pl.BlockSpec((BLK, D), lambda i: (i, 0)),
        out_shape=jax.ShapeDtypeStruct((NPAD, D), jnp.float32),
    )(acc1[:NPAD], acc1[NPAD:], y1, dinv, b1)


def _heads_kernel(acc2, y2, dinv, W2, b2):
    def body(a0_ref, a1_ref, y_ref, d_ref, w_ref, b_ref, mu_ref, ls_ref):
        g = (a0_ref[...] + a1_ref[...] - y_ref[...]) * d_ref[...]
        o = jnp.dot(g, w_ref[...], preferred_element_type=jnp.float32) + b_ref[...]
        mu_ref[...] = o[:, : D // 2]
        ls_ref[...] = o[:, D // 2 :]

    return pl.pallas_call(
        body,
        grid=(10,),
        in_specs=[
            pl.BlockSpec((BLK, D), lambda i: (i, 0)),
            pl.BlockSpec((BLK, D), lambda i: (i, 0)),
            pl.BlockSpec((BLK, D), lambda i: (i, 0)),
            pl.BlockSpec((BLK, 1), lambda i: (i, 0)),
            pl.BlockSpec((D, D), lambda i: (0, 0)),
            pl.BlockSpec((1, D), lambda i: (0, 0)),
        ],
        out_specs=[
            pl.BlockSpec((BLK, D // 2), lambda i: (i, 0)),
            pl.BlockSpec((BLK, D // 2), lambda i: (i, 0)),
        ],
        out_shape=[
            jax.ShapeDtypeStruct((NPAD, D // 2), jnp.float32),
            jax.ShapeDtypeStruct((NPAD, D // 2), jnp.float32),
        ],
    )(acc2[:NPAD], acc2[NPAD:], y2, dinv, W2, b2)


def kernel(x, edge_index, W1, b1, Wmu, bmu, Wls, bls):
    src = edge_index[0].astype(jnp.int32)
    dst = edge_index[1].astype(jnp.int32)
    pad = EPAD - E
    pad_lane = jnp.arange(pad, dtype=jnp.int32) % 128
    srcp = jnp.concatenate([src, pad_lane * 64])
    dstp = jnp.concatenate([dst, PAD_DST + pad_lane])
    src2d = srcp.reshape(ROWS, 128)
    dst2d = dstp.reshape(ROWS, 128)
    zeros2d = jnp.zeros((128, D), jnp.float32)
    iota2d = jnp.arange(128, dtype=jnp.int32).reshape(1, 128)
    xp = jnp.pad(x, ((0, NPAD - N), (0, 0)))
    W2 = jnp.concatenate([Wmu, Wls], axis=1)
    b2 = jnp.concatenate([bmu, bls]).reshape(1, D)
    b1r = b1.reshape(1, D)

    degp = _deg_kernel(dst2d, zeros2d, iota2d)
    deg0 = degp[:128].reshape(-1)[:NPAD].reshape(NPAD, 1)
    deg1 = degp[128:].reshape(-1)[:NPAD].reshape(NPAD, 1)
    y1, dinv = _lin1_scale_kernel(xp, W1, deg0, deg1)
    acc1 = _prop_kernel(y1, src2d, dst2d)
    y2 = _relu_scale_kernel(acc1, y1, dinv, b1r)
    acc2 = _prop_kernel(y2, src2d, dst2d)
    mu, logstd = _heads_kernel(acc2, y2, dinv, W2, b2)
    return (mu[:N], logstd[:N])


# speedup vs baseline: 33.9534x; 1.0115x over previous
import functools

import jax
import jax.numpy as jnp
from jax import lax
from jax.experimental import pallas as pl
from jax.experimental.pallas import tpu as pltpu
from jax.experimental.pallas import tpu_sc as plsc

N = 10000
E = 320000
NPAD = 10240
NPT = NPAD // 16
D = 128
NC = 2
NS = 16
EPAD = 327680
ROWS = EPAD // 128
CROWS = ROWS // NC
TROWS = CROWS // NS
CHUNKS = TROWS // 8
PAD_DST = N + 8
BLK = NPAD // 10
ICHUNK = NPT // 128


def _sc_mesh():
    return plsc.VectorSubcoreMesh(core_axis_name="c", subcore_axis_name="s")


def _deg_kernel(dst2d, zeros2d, iota2d):
    @functools.partial(
        pl.kernel,
        mesh=_sc_mesh(),
        out_type=jax.ShapeDtypeStruct((NC * 128, D), jnp.float32),
        scratch_types=[
            pltpu.VMEM((TROWS, 128), jnp.int32),
            pltpu.VMEM((1, 128), jnp.int32),
            pltpu.VMEM((128, D), jnp.float32),
            pltpu.VMEM((16, D), jnp.float32),
            pltpu.VMEM_SHARED((128, D), jnp.float32),
        ],
        compiler_params=pltpu.CompilerParams(needs_layout_passes=False),
    )
    def k(dst_h, z_h, iota_h, out_h, dst_v, idx_v, hist_v, wb_v, deg_sh):
        c = lax.axis_index("c")
        s = lax.axis_index("s")
        pltpu.sync_copy(z_h, hist_v)
        pltpu.sync_copy(iota_h, idx_v)

        @pl.when(s == 0)
        def _():
            pltpu.sync_copy(hist_v, deg_sh)

        plsc.subcore_barrier()

        row0 = c * CROWS + s * TROWS
        pltpu.sync_copy(dst_h.at[pl.ds(row0, TROWS)], dst_v)
        ones16 = jnp.full((16,), 1.0, dtype=jnp.float32)

        def chunk(kk, carry):
            for j in range(8):
                for t in range(8):
                    v = dst_v[kk * 8 + j, pl.ds(t * 16, 16)]
                    hi = jnp.right_shift(v, 7)
                    lo = jnp.bitwise_and(v, 127)
                    plsc.addupdate_scatter(hist_v, [hi, lo], ones16)
            return carry

        lax.fori_loop(0, CHUNKS, chunk, 0)
        pltpu.sync_copy(hist_v, deg_sh.at[idx_v.at[0]], add=True)
        plsc.subcore_barrier()

        @pl.when(s < 8)
        def _():
            pltpu.sync_copy(deg_sh.at[pl.ds(s * 16, 16)], wb_v)
            pltpu.sync_copy(wb_v, out_h.at[pl.ds(c * 128 + s * 16, 16)])

    return k(dst2d, zeros2d, iota2d)


def _prop_kernel(ytab, src2d, dst2d):
    @functools.partial(
        pl.kernel,
        mesh=_sc_mesh(),
        out_type=jax.ShapeDtypeStruct((2 * NPAD, D), jnp.float32),
        scratch_types=[
            pltpu.VMEM((16, 128), jnp.int32),
            pltpu.VMEM((16, 128), jnp.int32),
            pltpu.VMEM((256, D), jnp.float32),
            pltpu.VMEM_SHARED((NPAD, D), jnp.float32),
            pltpu.SemaphoreType.DMA,
            pltpu.SemaphoreType.DMA,
        ],
    )
    def k(ytab_h, src_h, dst_h, out_h, src_v, dst_v, rows_v, acc_sh, sem, isem):
        c = lax.axis_index("c")
        s = lax.axis_index("s")
        nbase = s * NPT
        for t in range(ICHUNK):
            pltpu.sync_copy(
                ytab_h.at[pl.ds(nbase + t * 128, 128)], rows_v.at[pl.ds(0, 128)]
            )
            pltpu.sync_copy(
                rows_v.at[pl.ds(0, 128)], acc_sh.at[pl.ds(nbase + t * 128, 128)]
            )
        plsc.subcore_barrier()

        row0 = c * CROWS + s * TROWS
        ihs = [None] * (CHUNKS * 2)
        ihs[0] = pltpu.async_copy(src_h.at[pl.ds(row0, 8)], src_v.at[pl.ds(0, 8)], isem)
        ihs[1] = pltpu.async_copy(dst_h.at[pl.ds(row0, 8)], dst_v.at[pl.ds(0, 8)], isem)
        hs = [None] * TROWS
        for kk in range(CHUNKS):
            ib = (kk % 2) * 8
            if kk + 1 < CHUNKS:
                nb = ((kk + 1) % 2) * 8
                r = row0 + (kk + 1) * 8
                ihs[2 * kk + 2] = pltpu.async_copy(
                    src_h.at[pl.ds(r, 8)], src_v.at[pl.ds(nb, 8)], isem
                )
                ihs[2 * kk + 3] = pltpu.async_copy(
                    dst_h.at[pl.ds(r, 8)], dst_v.at[pl.ds(nb, 8)], isem
                )
            ihs[2 * kk].wait()
            ihs[2 * kk + 1].wait()
            for j in range(8):
                g = kk * 8 + j
                if j == 0:
                    hs[g] = pltpu.async_copy(
                        ytab_h.at[src_v.at[ib]], rows_v.at[pl.ds((g % 2) * 128, 128)], sem
                    )
                if g + 1 < TROWS and j < 7:
                    nsl = ((g + 1) % 2) * 128
                    hs[g + 1] = pltpu.async_copy(
                        ytab_h.at[src_v.at[ib + j + 1]], rows_v.at[pl.ds(nsl, 128)], sem
                    )
                hs[g].wait()
                pltpu.sync_copy(
                    rows_v.at[pl.ds((g % 2) * 128, 128)],
                    acc_sh.at[dst_v.at[ib + j]],
                    add=True,
                )
        plsc.subcore_barrier()
        for t in range(ICHUNK):
            pltpu.sync_copy(
                acc_sh.at[pl.ds(nbase + t * 128, 128)], rows_v.at[pl.ds(0, 128)]
            )
            pltpu.sync_copy(
                rows_v.at[pl.ds(0, 128)],
                out_h.at[pl.ds(c * NPAD + nbase + t * 128, 128)],
            )

    return k(ytab, src2d, dst2d)


def _lin1_scale_kernel(x, W1, deg0, deg1):
    def body(x_ref, w_ref, d0_ref, d1_ref, y_ref, dv_ref):
        deg = d0_ref[...] + d1_ref[...]
        dinv = lax.rsqrt(deg + 1.0)
        dv_ref[...] = dinv
        lin = jnp.dot(x_ref[...], w_ref[...], preferred_element_type=jnp.float32)
        y_ref[...] = lin * dinv

    return pl.pallas_call(
        body,
        grid=(10,),
        in_specs=[
            pl.BlockSpec((BLK, D), lambda i: (i, 0)),
            pl.BlockSpec((D, D), lambda i: (0, 0)),
            pl.BlockSpec((BLK, 1), lambda i: (i, 0)),
            pl.BlockSpec((BLK, 1), lambda i: (i, 0)),
        ],
        out_specs=[
            pl.BlockSpec((BLK, D), lambda i: (i, 0)),
            pl.BlockSpec((BLK, 1), lambda i: (i, 0)),
        ],
        out_shape=[
            jax.ShapeDtypeStruct((NPAD, D), jnp.float32),
            jax.ShapeDtypeStruct((NPAD, 1), jnp.float32),
        ],
    )(x, W1, deg0, deg1)


def _relu_scale_kernel(acc1, y1, dinv, b1):
    def body(a0_ref, a1_ref, y_ref, d_ref, b_ref, o_ref):
        dcol = d_ref[...]
        acc = a0_ref[...] + a1_ref[...] - y_ref[...]
        h = jnp.maximum(acc * dcol + b_ref[...], 0.0)
        o_ref[...] = h * dcol

    return pl.pallas_call(
        body,
        grid=(10,),
        in_specs=[
            pl.BlockSpec((BLK, D), lambda i: (i, 0)),
            pl.BlockSpec((BLK, D), lambda i: (i, 0)),
            pl.BlockSpec((BLK, D), lambda i: (i, 0)),
            pl.BlockSpec((BLK, 1), lambda i: (i, 0)),
            pl.BlockSpec((1, D), lambda i: (0, 0)),
        ],
        out_specs=pl.BlockSpec((BLK, D), lambda i: (i, 0)),
        out_shape=jax.ShapeDtypeStruct((NPAD, D), jnp.float32),
    )(acc1[:NPAD], acc1[NPAD:], y1, dinv, b1)


def _heads_kernel(acc2, y2, dinv, W2, b2):
    def body(a0_ref, a1_ref, y_ref, d_ref, w_ref, b_ref, mu_ref, ls_ref):
        g = (a0_ref[...] + a1_ref[...] - y_ref[...]) * d_ref[...]
        o = jnp.dot(g, w_ref[...], preferred_element_type=jnp.float32) + b_ref[...]
        mu_ref[...] = o[:, : D // 2]
        ls_ref[...] = o[:, D // 2 :]

    return pl.pallas_call(
        body,
        grid=(10,),
        in_specs=[
            pl.BlockSpec((BLK, D), lambda i: (i, 0)),
            pl.BlockSpec((BLK, D), lambda i: (i, 0)),
            pl.BlockSpec((BLK, D), lambda i: (i, 0)),
            pl.BlockSpec((BLK, 1), lambda i: (i, 0)),
            pl.BlockSpec((D, D), lambda i: (0, 0)),
            pl.BlockSpec((1, D), lambda i: (0, 0)),
        ],
        out_specs=[
            pl.BlockSpec((BLK, D // 2), lambda i: (i, 0)),
            pl.BlockSpec((BLK, D // 2), lambda i: (i, 0)),
        ],
        out_shape=[
            jax.ShapeDtypeStruct((N, D // 2), jnp.float32),
            jax.ShapeDtypeStruct((N, D // 2), jnp.float32),
        ],
    )(acc2[:NPAD], acc2[NPAD:], y2, dinv, W2, b2)


def kernel(x, edge_index, W1, b1, Wmu, bmu, Wls, bls):
    src = edge_index[0].astype(jnp.int32)
    dst = edge_index[1].astype(jnp.int32)
    pad = EPAD - E
    pad_lane = jnp.arange(pad, dtype=jnp.int32) % 128
    srcp = jnp.concatenate([src, pad_lane * 64])
    dstp = jnp.concatenate([dst, PAD_DST + pad_lane])
    src2d = srcp.reshape(ROWS, 128)
    dst2d = dstp.reshape(ROWS, 128)
    zeros2d = jnp.zeros((128, D), jnp.float32)
    iota2d = jnp.arange(128, dtype=jnp.int32).reshape(1, 128)
    W2 = jnp.concatenate([Wmu, Wls], axis=1)
    b2 = jnp.concatenate([bmu, bls]).reshape(1, D)
    b1r = b1.reshape(1, D)

    degp = _deg_kernel(dst2d, zeros2d, iota2d)
    deg0 = degp[:128].reshape(-1)[:NPAD].reshape(NPAD, 1)
    deg1 = degp[128:].reshape(-1)[:NPAD].reshape(NPAD, 1)
    y1, dinv = _lin1_scale_kernel(x, W1, deg0, deg1)
    acc1 = _prop_kernel(y1, src2d, dst2d)
    y2 = _relu_scale_kernel(acc1, y1, dinv, b1r)
    acc2 = _prop_kernel(y2, src2d, dst2d)
    mu, logstd = _heads_kernel(acc2, y2, dinv, W2, b2)
    return (mu, logstd)


# speedup vs baseline: 34.8683x; 1.0269x over previous
import functools

import jax
import jax.numpy as jnp
from jax import lax
from jax.experimental import pallas as pl
from jax.experimental.pallas import tpu as pltpu
from jax.experimental.pallas import tpu_sc as plsc

N = 10000
E = 320000
NPAD = 10240
NPT = NPAD // 16
D = 128
NC = 2
NS = 16
EPAD = 327680
ROWS = EPAD // 128
CROWS = ROWS // NC
TROWS = CROWS // NS
CHUNKS = TROWS // 8
PAD_DST = N + 8
BLK = NPAD // 10
ICHUNK = NPT // 128


def _sc_mesh():
    return plsc.VectorSubcoreMesh(core_axis_name="c", subcore_axis_name="s")


def _deg_kernel(dst2d, zeros2d, iota2d):
    @functools.partial(
        pl.kernel,
        mesh=_sc_mesh(),
        out_type=jax.ShapeDtypeStruct((NC * 128, D), jnp.float32),
        scratch_types=[
            pltpu.VMEM((TROWS, 128), jnp.int32),
            pltpu.VMEM((1, 128), jnp.int32),
            pltpu.VMEM((128, D), jnp.float32),
            pltpu.VMEM((16, D), jnp.float32),
            pltpu.VMEM_SHARED((128, D), jnp.float32),
        ],
        compiler_params=pltpu.CompilerParams(needs_layout_passes=False),
    )
    def k(dst_h, z_h, iota_h, out_h, dst_v, idx_v, hist_v, wb_v, deg_sh):
        c = lax.axis_index("c")
        s = lax.axis_index("s")
        pltpu.sync_copy(z_h, hist_v)
        pltpu.sync_copy(iota_h, idx_v)

        @pl.when(s == 0)
        def _():
            pltpu.sync_copy(hist_v, deg_sh)

        plsc.subcore_barrier()

        row0 = c * CROWS + s * TROWS
        pltpu.sync_copy(dst_h.at[pl.ds(row0, TROWS)], dst_v)
        ones16 = jnp.full((16,), 1.0, dtype=jnp.float32)

        def chunk(kk, carry):
            for j in range(8):
                for t in range(8):
                    v = dst_v[kk * 8 + j, pl.ds(t * 16, 16)]
                    hi = jnp.right_shift(v, 7)
                    lo = jnp.bitwise_and(v, 127)
                    plsc.addupdate_scatter(hist_v, [hi, lo], ones16)
            return carry

        lax.fori_loop(0, CHUNKS, chunk, 0)
        pltpu.sync_copy(hist_v, deg_sh.at[idx_v.at[0]], add=True)
        plsc.subcore_barrier()

        @pl.when(s < 8)
        def _():
            pltpu.sync_copy(deg_sh.at[pl.ds(s * 16, 16)], wb_v)
            pltpu.sync_copy(wb_v, out_h.at[pl.ds(c * 128 + s * 16, 16)])

    return k(dst2d, zeros2d, iota2d)


def _prop_kernel(ytab, zeros2d, src2d, dst2d):
    @functools.partial(
        pl.kernel,
        mesh=_sc_mesh(),
        out_type=jax.ShapeDtypeStruct((2 * NPAD, D), jnp.float32),
        scratch_types=[
            pltpu.VMEM((16, 128), jnp.int32),
            pltpu.VMEM((16, 128), jnp.int32),
            pltpu.VMEM((256, D), jnp.float32),
            pltpu.VMEM_SHARED((NPAD, D), jnp.float32),
            pltpu.SemaphoreType.DMA,
            pltpu.SemaphoreType.DMA,
        ],
    )
    def k(ytab_h, z_h, src_h, dst_h, out_h, src_v, dst_v, rows_v, acc_sh, sem, isem):
        c = lax.axis_index("c")
        s = lax.axis_index("s")
        nbase = s * NPT
        pltpu.sync_copy(z_h, rows_v.at[pl.ds(0, 128)])
        for t in range(ICHUNK):
            pltpu.sync_copy(
                rows_v.at[pl.ds(0, 128)], acc_sh.at[pl.ds(nbase + t * 128, 128)]
            )
        plsc.subcore_barrier()

        row0 = c * CROWS + s * TROWS
        ihs = [None] * (CHUNKS * 2)
        ihs[0] = pltpu.async_copy(src_h.at[pl.ds(row0, 8)], src_v.at[pl.ds(0, 8)], isem)
        ihs[1] = pltpu.async_copy(dst_h.at[pl.ds(row0, 8)], dst_v.at[pl.ds(0, 8)], isem)
        hs = [None] * TROWS
        for kk in range(CHUNKS):
            ib = (kk % 2) * 8
            if kk + 1 < CHUNKS:
                nb = ((kk + 1) % 2) * 8
                r = row0 + (kk + 1) * 8
                ihs[2 * kk + 2] = pltpu.async_copy(
                    src_h.at[pl.ds(r, 8)], src_v.at[pl.ds(nb, 8)], isem
                )
                ihs[2 * kk + 3] = pltpu.async_copy(
                    dst_h.at[pl.ds(r, 8)], dst_v.at[pl.ds(nb, 8)], isem
                )
            ihs[2 * kk].wait()
            ihs[2 * kk + 1].wait()
            for j in range(8):
                g = kk * 8 + j
                if j == 0:
                    hs[g] = pltpu.async_copy(
                        ytab_h.at[src_v.at[ib]], rows_v.at[pl.ds((g % 2) * 128, 128)], sem
                    )
                if g + 1 < TROWS and j < 7:
                    nsl = ((g + 1) % 2) * 128
                    hs[g + 1] = pltpu.async_copy(
                        ytab_h.at[src_v.at[ib + j + 1]], rows_v.at[pl.ds(nsl, 128)], sem
                    )
                hs[g].wait()
                pltpu.sync_copy(
                    rows_v.at[pl.ds((g % 2) * 128, 128)],
                    acc_sh.at[dst_v.at[ib + j]],
                    add=True,
                )
        plsc.subcore_barrier()
        for t in range(ICHUNK):
            pltpu.sync_copy(
                acc_sh.at[pl.ds(nbase + t * 128, 128)], rows_v.at[pl.ds(0, 128)]
            )
            pltpu.sync_copy(
                rows_v.at[pl.ds(0, 128)],
                out_h.at[pl.ds(c * NPAD + nbase + t * 128, 128)],
            )

    return k(ytab, zeros2d, src2d, dst2d)


def _lin1_scale_kernel(x, W1, deg0, deg1):
    def body(x_ref, w_ref, d0_ref, d1_ref, y_ref, dv_ref):
        deg = d0_ref[...] + d1_ref[...]
        dinv = lax.rsqrt(deg + 1.0)
        dv_ref[...] = dinv
        lin = jnp.dot(x_ref[...], w_ref[...], preferred_element_type=jnp.float32)
        y_ref[...] = lin * dinv

    return pl.pallas_call(
        body,
        grid=(10,),
        in_specs=[
            pl.BlockSpec((BLK, D), lambda i: (i, 0)),
            pl.BlockSpec((D, D), lambda i: (0, 0)),
            pl.BlockSpec((BLK, 1), lambda i: (i, 0)),
            pl.BlockSpec((BLK, 1), lambda i: (i, 0)),
        ],
        out_specs=[
            pl.BlockSpec((BLK, D), lambda i: (i, 0)),
            pl.BlockSpec((BLK, 1), lambda i: (i, 0)),
        ],
        out_shape=[
            jax.ShapeDtypeStruct((NPAD, D), jnp.float32),
            jax.ShapeDtypeStruct((NPAD, 1), jnp.float32),
        ],
    )(x, W1, deg0, deg1)


def _relu_scale_kernel(acc1, y1, dinv, b1):
    def body(a0_ref, a1_ref, y_ref, d_ref, b_ref, o_ref):
        dcol = d_ref[...]
        acc = a0_ref[...] + a1_ref[...] + y_ref[...]
        h = jnp.maximum(acc * dcol + b_ref[...], 0.0)
        o_ref[...] = h * dcol

    return pl.pallas_call(
        body,
        grid=(10,),
        in_specs=[
            pl.BlockSpec((BLK, D), lambda i: (i, 0)),
            pl.BlockSpec((BLK, D), lambda i: (i, 0)),
            pl.BlockSpec((BLK, D), lambda i: (i, 0)),
            pl.BlockSpec((BLK, 1), lambda i: (i, 0)),
            pl.BlockSpec((1, D), lambda i: (0, 0)),
        ],
        out_specs=pl.BlockSpec((BLK, D), lambda i: (i, 0)),
        out_shape=jax.ShapeDtypeStruct((NPAD, D), jnp.float32),
    )(acc1[:NPAD], acc1[NPAD:], y1, dinv, b1)


def _heads_kernel(acc2, y2, dinv, W2, b2):
    def body(a0_ref, a1_ref, y_ref, d_ref, w_ref, b_ref, mu_ref, ls_ref):
        g = (a0_ref[...] + a1_ref[...] + y_ref[...]) * d_ref[...]
        o = jnp.dot(g, w_ref[...], preferred_element_type=jnp.float32) + b_ref[...]
        mu_ref[...] = o[:, : D // 2]
        ls_ref[...] = o[:, D // 2 :]

    return pl.pallas_call(
        body,
        grid=(10,),
        in_specs=[
            pl.BlockSpec((BLK, D), lambda i: (i, 0)),
            pl.BlockSpec((BLK, D), lambda i: (i, 0)),
            pl.BlockSpec((BLK, D), lambda i: (i, 0)),
            pl.BlockSpec((BLK, 1), lambda i: (i, 0)),
            pl.BlockSpec((D, D), lambda i: (0, 0)),
            pl.BlockSpec((1, D), lambda i: (0, 0)),
        ],
        out_specs=[
            pl.BlockSpec((BLK, D // 2), lambda i: (i, 0)),
            pl.BlockSpec((BLK, D // 2), lambda i: (i, 0)),
        ],
        out_shape=[
            jax.ShapeDtypeStruct((N, D // 2), jnp.float32),
            jax.ShapeDtypeStruct((N, D // 2), jnp.float32),
        ],
    )(acc2[:NPAD], acc2[NPAD:], y2, dinv, W2, b2)


def kernel(x, edge_index, W1, b1, Wmu, bmu, Wls, bls):
    src = edge_index[0].astype(jnp.int32)
    dst = edge_index[1].astype(jnp.int32)
    pad = EPAD - E
    pad_lane = jnp.arange(pad, dtype=jnp.int32) % 128
    srcp = jnp.concatenate([src, pad_lane * 64])
    dstp = jnp.concatenate([dst, PAD_DST + pad_lane])
    src2d = srcp.reshape(ROWS, 128)
    dst2d = dstp.reshape(ROWS, 128)
    zeros2d = jnp.zeros((128, D), jnp.float32)
    iota2d = jnp.arange(128, dtype=jnp.int32).reshape(1, 128)
    W2 = jnp.concatenate([Wmu, Wls], axis=1)
    b2 = jnp.concatenate([bmu, bls]).reshape(1, D)
    b1r = b1.reshape(1, D)

    degp = _deg_kernel(dst2d, zeros2d, iota2d)
    deg0 = degp[:128].reshape(-1)[:NPAD].reshape(NPAD, 1)
    deg1 = degp[128:].reshape(-1)[:NPAD].reshape(NPAD, 1)
    y1, dinv = _lin1_scale_kernel(x, W1, deg0, deg1)
    acc1 = _prop_kernel(y1, zeros2d, src2d, dst2d)
    y2 = _relu_scale_kernel(acc1, y1, dinv, b1r)
    acc2 = _prop_kernel(y2, zeros2d, src2d, dst2d)
    mu, logstd = _heads_kernel(acc2, y2, dinv, W2, b2)
    return (mu, logstd)
